# Initial kernel scaffold; baseline (speedup 1.0000x reference)
import functools

import jax
import jax.numpy as jnp
from jax import lax
from jax.experimental import pallas as pl
from jax.experimental.pallas import tpu as pltpu
from jax.experimental.pallas import tpu_sc as plsc

N = 10000
E = 160000
D = 128
DP = 144
NT = 16
EPT = E // NT
CHUNK = 125
NCHUNK = EPT // CHUNK
RPT = N // NT
BM = 1000


def _mm_body(h_ref, w_ref, o_ref):
    acc = jnp.dot(h_ref[...], w_ref[0],
                  preferred_element_type=jnp.float32,
                  precision=lax.Precision.HIGHEST)
    col = lax.broadcasted_iota(jnp.int32, (BM, DP), 1)
    o_ref[...] = acc + jnp.where(col == D, 1.0, 0.0)


def _fin_body(a0_ref, a1_ref, o_ref):
    x0 = a0_ref[...]
    x1 = a1_ref[...]
    d0 = jnp.maximum(x0[:, D:D + 1], 1.0)
    d1 = jnp.maximum(x1[:, D:D + 1], 1.0)
    o_ref[...] = jnp.maximum(x0[:, :D] / d0 + x1[:, :D] / d1, 0.0)


@functools.partial(
    pl.kernel,
    out_type=jax.ShapeDtypeStruct((2 * N, DP), jnp.float32),
    mesh=plsc.VectorSubcoreMesh(core_axis_name="c", subcore_axis_name="s"),
    scratch_types=[
        pltpu.VMEM((NCHUNK, CHUNK), jnp.int32),
        pltpu.VMEM((NCHUNK, CHUNK), jnp.int32),
        pltpu.VMEM((2, CHUNK, DP), jnp.float32),
        pltpu.VMEM_SHARED((N, DP), jnp.float32),
        pltpu.SemaphoreType.DMA,
    ],
)
def _sc_aggregate(hwp_hbm, src_hbm, dst_hbm, out_hbm,
                  src_v, dst_v, rows_v, agg_s, gsem):
    c = lax.axis_index("c")
    s = lax.axis_index("s")

    pltpu.sync_copy(src_hbm.at[c, s], src_v)
    pltpu.sync_copy(dst_hbm.at[c, s], dst_v)

    zrow = jnp.zeros((16,), jnp.float32)

    def zbody(i, carry):
        for k in range(DP // 16):
            rows_v[0, i, pl.ds(k * 16, 16)] = zrow
        return carry

    lax.fori_loop(0, CHUNK, zbody, 0)
    for j in range(RPT // CHUNK):
        pltpu.sync_copy(rows_v.at[0], agg_s.at[pl.ds(s * RPT + j * CHUNK, CHUNK)])
    plsc.subcore_barrier()

    def body(j, carry):
        pltpu.async_copy(hwp_hbm.at[src_v.at[j]], rows_v.at[0], gsem).wait()
        pltpu.sync_copy(rows_v.at[0], agg_s.at[dst_v.at[j]], add=True)
        return carry

    lax.fori_loop(0, NCHUNK, body, 0)
    plsc.subcore_barrier()

    pltpu.sync_copy(agg_s.at[pl.ds(s * RPT, RPT)],
                    out_hbm.at[pl.ds(c * N + s * RPT, RPT)])


def kernel(inp_h, edge_index_e0, edge_index_e1, W_e0, W_e1):
    src = jnp.stack([edge_index_e0[0], edge_index_e1[0] + N])
    src = src.reshape(2, NT, NCHUNK, CHUNK)
    dst = jnp.stack([edge_index_e0[1], edge_index_e1[1]])
    dst = dst.reshape(2, NT, NCHUNK, CHUNK)
    wp = jnp.zeros((2, D, DP), jnp.float32)
    wp = wp.at[:, :, :D].set(jnp.stack([W_e0, W_e1]))

    hwp = pl.pallas_call(
        _mm_body,
        grid=(2, N // BM),
        in_specs=[
            pl.BlockSpec((BM, D), lambda r, i: (i, 0)),
            pl.BlockSpec((1, D, DP), lambda r, i: (r, 0, 0)),
        ],
        out_specs=pl.BlockSpec((BM, DP), lambda r, i: (r * (N // BM) + i, 0)),
        out_shape=jax.ShapeDtypeStruct((2 * N, DP), jnp.float32),
    )(inp_h, wp)

    agg = _sc_aggregate(hwp, src, dst)

    out = pl.pallas_call(
        _fin_body,
        grid=(N // BM,),
        in_specs=[
            pl.BlockSpec((BM, DP), lambda i: (i, 0)),
            pl.BlockSpec((BM, DP), lambda i: (i + N // BM, 0)),
        ],
        out_specs=pl.BlockSpec((BM, D), lambda i: (i, 0)),
        out_shape=jax.ShapeDtypeStruct((N, D), jnp.float32),
    )(agg, agg)
    return out


# speedup vs baseline: 4.8801x; 4.8801x over previous
import functools

import jax
import jax.numpy as jnp
from jax import lax
from jax.experimental import pallas as pl
from jax.experimental.pallas import tpu as pltpu
from jax.experimental.pallas import tpu_sc as plsc

N = 10000
E = 160000
D = 128
DP = 144
NT = 16
EPT = E // NT
CHUNK = 100
NCHUNK = EPT // CHUNK
RPT = N // NT
BM = 1000


def _mm_body(h_ref, w_ref, o_ref):
    acc = jnp.dot(h_ref[...], w_ref[0],
                  preferred_element_type=jnp.float32,
                  precision=lax.Precision.HIGHEST)
    col = lax.broadcasted_iota(jnp.int32, (BM, DP), 1)
    o_ref[...] = acc + jnp.where(col == D, 1.0, 0.0)


def _fin_body(a0_ref, a1_ref, o_ref):
    x0 = a0_ref[...]
    x1 = a1_ref[...]
    d0 = jnp.maximum(x0[:, D:D + 1], 1.0)
    d1 = jnp.maximum(x1[:, D:D + 1], 1.0)
    o_ref[...] = jnp.maximum(x0[:, :D] / d0 + x1[:, :D] / d1, 0.0)


@functools.partial(
    pl.kernel,
    out_type=jax.ShapeDtypeStruct((2 * N, DP), jnp.float32),
    mesh=plsc.VectorSubcoreMesh(core_axis_name="c", subcore_axis_name="s"),
    scratch_types=[
        pltpu.VMEM((CHUNK,), jnp.int32),
        pltpu.VMEM((CHUNK,), jnp.int32),
        pltpu.VMEM((2, CHUNK, DP), jnp.float32),
        pltpu.VMEM_SHARED((N, DP), jnp.float32),
        pltpu.SemaphoreType.DMA,
    ],
    compiler_params=pltpu.CompilerParams(use_tc_tiling_on_sc=False),
)
def _sc_aggregate(hwp_hbm, src_hbm, dst_hbm, out_hbm,
                  src_v, dst_v, rows_v, agg_s, gsem):
    c = lax.axis_index("c")
    s = lax.axis_index("s")

    zrow = jnp.zeros((16,), jnp.float32)

    def zbody(i, carry):
        for k in range(DP // 16):
            rows_v[0, i, pl.ds(k * 16, 16)] = zrow
        return carry

    lax.fori_loop(0, CHUNK, zbody, 0)
    for j in range(RPT // CHUNK):
        pltpu.sync_copy(rows_v.at[0], agg_s.at[pl.ds(s * RPT + j * CHUNK, CHUNK)])
    plsc.subcore_barrier()

    def body(j, carry):
        pltpu.sync_copy(src_hbm.at[c, s, j], src_v)
        pltpu.sync_copy(dst_hbm.at[c, s, j], dst_v)
        pltpu.async_copy(hwp_hbm.at[src_v], rows_v.at[0], gsem).wait()
        pltpu.sync_copy(rows_v.at[0], agg_s.at[dst_v], add=True)
        return carry

    lax.fori_loop(0, NCHUNK, body, 0)
    plsc.subcore_barrier()

    pltpu.sync_copy(agg_s.at[pl.ds(s * RPT, RPT)],
                    out_hbm.at[pl.ds(c * N + s * RPT, RPT)])


def kernel(inp_h, edge_index_e0, edge_index_e1, W_e0, W_e1):
    src = jnp.stack([edge_index_e0[0], edge_index_e1[0] + N])
    src = src.reshape(2, NT, NCHUNK, CHUNK)
    dst = jnp.stack([edge_index_e0[1], edge_index_e1[1]])
    dst = dst.reshape(2, NT, NCHUNK, CHUNK)
    wp = jnp.zeros((2, D, DP), jnp.float32)
    wp = wp.at[:, :, :D].set(jnp.stack([W_e0, W_e1]))

    hwp = pl.pallas_call(
        _mm_body,
        grid=(2, N // BM),
        in_specs=[
            pl.BlockSpec((BM, D), lambda r, i: (i, 0)),
            pl.BlockSpec((1, D, DP), lambda r, i: (r, 0, 0)),
        ],
        out_specs=pl.BlockSpec((BM, DP), lambda r, i: (r * (N // BM) + i, 0)),
        out_shape=jax.ShapeDtypeStruct((2 * N, DP), jnp.float32),
    )(inp_h, wp)

    agg = _sc_aggregate(hwp, src, dst)

    out = pl.pallas_call(
        _fin_body,
        grid=(N // BM,),
        in_specs=[
            pl.BlockSpec((BM, DP), lambda i: (i, 0)),
            pl.BlockSpec((BM, DP), lambda i: (i + N // BM, 0)),
        ],
        out_specs=pl.BlockSpec((BM, D), lambda i: (i, 0)),
        out_shape=jax.ShapeDtypeStruct((N, D), jnp.float32),
    )(agg, agg)
    return out
